# baseline (speedup 1.0000x reference)
import math

import jax
import jax.numpy as jnp
from jax.experimental import pallas as pl
from jax.experimental.pallas import tpu as pltpu

_BLOCK = 3584
_CHUNK = 896


def _nn_dist_kernel(ea_ref, eb_ref, ct_ref, o_ref):
    ctf = ct_ref[...]
    ctb = ctf.astype(jnp.bfloat16)
    cnorm = 0.25 * jnp.sum(ctf * ctf, axis=0, keepdims=True)
    cnormb = cnorm.astype(jnp.bfloat16)
    half_rows = _BLOCK // 128
    rows = _CHUNK // 128
    for h, e_ref in enumerate((ea_ref, eb_ref)):
        for k in range(_BLOCK // _CHUNK):
            ef = e_ref[pl.ds(k * _CHUNK, _CHUNK), :]
            eb = ef.astype(jnp.bfloat16)
            enorm = jnp.sum(ef * ef, axis=1)
            prod = jax.lax.dot_general(
                eb, ctb,
                dimension_numbers=(((1,), (0,)), ((), ())),
                preferred_element_type=jnp.float32,
            )
            dmin = jnp.min(cnormb + prod.astype(jnp.bfloat16),
                           axis=1).astype(jnp.float32)
            o_ref[0, pl.ds(h * half_rows + k * rows, rows), :] = jnp.sqrt(
                enorm + dmin).reshape(rows, 128)


def kernel(embeds, centroids):
    B, N, D = embeds.shape
    P = centroids.shape[0]
    M = B * N
    half = M // 2
    e2 = embeds.reshape(M, D)
    ct = (-2.0 * centroids).T
    n_tiles = half // _BLOCK
    rows_out = 2 * (_BLOCK // 128)

    out = pl.pallas_call(
        _nn_dist_kernel,
        grid=(n_tiles,),
        in_specs=[
            pl.BlockSpec((_BLOCK, D), lambda i: (i, 0)),
            pl.BlockSpec((_BLOCK, D), lambda i: (i + n_tiles, 0)),
            pl.BlockSpec((D, P), lambda i: (0, 0)),
        ],
        out_specs=pl.BlockSpec((1, rows_out, 128), lambda i: (i, 0, 0)),
        out_shape=jax.ShapeDtypeStruct((n_tiles, rows_out, 128), jnp.float32),
        compiler_params=pltpu.CompilerParams(dimension_semantics=("parallel",)),
    )(e2, e2, ct)

    h = int(math.sqrt(N))
    a = out[:, : rows_out // 2, :].reshape(half)
    b = out[:, rows_out // 2 :, :].reshape(half)
    score = jnp.concatenate([a, b]).reshape(B, 1, h, h)
    loss = jnp.zeros(())
    return (loss, score)


# speedup vs baseline: 1.0048x; 1.0048x over previous
import math

import jax
import jax.numpy as jnp
from jax.experimental import pallas as pl
from jax.experimental.pallas import tpu as pltpu

_BLOCK = 7168
_CHUNK = 896


def _nn_dist_kernel(e_ref, ct_ref, o_ref):
    ctf = ct_ref[...]
    ctb = ctf.astype(jnp.bfloat16)
    cnorm = 0.25 * jnp.sum(ctf * ctf, axis=0, keepdims=True)
    cnormb = cnorm.astype(jnp.bfloat16)
    rows = _CHUNK // 128
    for k in range(_BLOCK // _CHUNK):
        ef = e_ref[pl.ds(k * _CHUNK, _CHUNK), :]
        eb = ef.astype(jnp.bfloat16)
        enorm = jnp.sum(ef * ef, axis=1)
        prod = jax.lax.dot_general(
            eb, ctb,
            dimension_numbers=(((1,), (0,)), ((), ())),
            preferred_element_type=jnp.float32,
        )
        dmin = jnp.min(cnormb + prod.astype(jnp.bfloat16),
                       axis=1).astype(jnp.float32)
        o_ref[0, pl.ds(k * rows, rows), :] = jnp.sqrt(enorm + dmin).reshape(
            rows, 128)


def kernel(embeds, centroids):
    B, N, D = embeds.shape
    P = centroids.shape[0]
    M = B * N
    e2 = embeds.reshape(M, D)
    ct = (-2.0 * centroids).T
    n_tiles = M // _BLOCK
    rows_out = _BLOCK // 128

    out = pl.pallas_call(
        _nn_dist_kernel,
        grid=(n_tiles,),
        in_specs=[
            pl.BlockSpec((_BLOCK, D), lambda i: (i, 0)),
            pl.BlockSpec((D, P), lambda i: (0, 0)),
        ],
        out_specs=pl.BlockSpec((1, rows_out, 128), lambda i: (i, 0, 0)),
        out_shape=jax.ShapeDtypeStruct((n_tiles, rows_out, 128), jnp.float32),
        compiler_params=pltpu.CompilerParams(dimension_semantics=("parallel",)),
    )(e2, ct)

    h = int(math.sqrt(N))
    score = out.reshape(B, 1, h, h)
    loss = jnp.zeros(())
    return (loss, score)
